# Initial kernel scaffold; baseline (speedup 1.0000x reference)
import jax
import jax.numpy as jnp
from jax.experimental import pallas as pl


def kernel(x):
    raise NotImplementedError("write your pallas kernel here")


# speedup vs baseline: 8.6102x; 8.6102x over previous
import jax
import jax.numpy as jnp
from jax.experimental import pallas as pl

CONF_THRES = 0.25
IOU_THRES = 0.45
MAX_NMS = 1000
MAX_DET = 300
MAX_WH = 4096.0

_N = 1024
_R = 512


def _scores_body(pred_ref, scores_ref, cls_ref):
    blk = pred_ref[0]
    rows = blk.shape[0]
    obj = blk[:, 4:5]
    prod = blk * obj
    lane = jax.lax.broadcasted_iota(jnp.int32, (rows, 85), 1)
    masked = jnp.where(lane >= 5, prod, -jnp.inf)
    conf = jnp.max(masked, axis=1, keepdims=True)
    cand = jnp.where(masked >= conf,
                     lane.astype(jnp.float32), 1e9)
    cls_id = jnp.min(cand, axis=1) - 5.0
    confv = conf[:, 0]
    scores_ref[0, 0, :] = jnp.where(confv > CONF_THRES, confv, -1.0)
    cls_ref[0, 0, :] = cls_id


def _compute_scores(pred):
    B, N, C = pred.shape
    rows = 4000
    nr = N // rows
    grid = (B, nr)
    scores, cls_id = pl.pallas_call(
        _scores_body,
        grid=grid,
        in_specs=[pl.BlockSpec((1, rows, C), lambda b, r: (b, r, 0))],
        out_specs=[pl.BlockSpec((1, 1, rows), lambda b, r: (b * nr + r, 0, 0)),
                   pl.BlockSpec((1, 1, rows), lambda b, r: (b * nr + r, 0, 0))],
        out_shape=[jax.ShapeDtypeStruct((B * nr, 1, rows), jnp.float32),
                   jax.ShapeDtypeStruct((B * nr, 1, rows), jnp.float32)],
    )(pred)
    return scores.reshape(B, N), cls_id.reshape(B, N)


def _nms_body(d_ref, dt_ref, out_ref):
    d = d_ref[0]
    dt = dt_ref[0]

    xr, yr, wr, hr = d[0:1, :], d[1:2, :], d[2:3, :], d[3:4, :]
    scr, clr = d[4:5, :], d[5:6, :]
    offr = clr * MAX_WH
    rx1 = (xr - wr / 2.0) + offr
    ry1 = (yr - hr / 2.0) + offr
    rx2 = (xr + wr / 2.0) + offr
    ry2 = (yr + hr / 2.0) + offr
    area_r = (rx2 - rx1) * (ry2 - ry1)

    xc, yc, wc, hc = dt[:, 0:1], dt[:, 1:2], dt[:, 2:3], dt[:, 3:4]
    clc = dt[:, 5:6]
    offc = clc * MAX_WH
    cx1 = (xc - wc / 2.0) + offc
    cy1 = (yc - hc / 2.0) + offc
    cx2 = (xc + wc / 2.0) + offc
    cy2 = (yc + hc / 2.0) + offc
    area_c = (cx2 - cx1) * (cy2 - cy1)

    iw = jnp.clip(jnp.minimum(cx2, rx2) - jnp.maximum(cx1, rx1), 0.0, None)
    ih = jnp.clip(jnp.minimum(cy2, ry2) - jnp.maximum(cy1, ry1), 0.0, None)
    inter = iw * ih
    iou = inter / (area_c + area_r - inter + 1e-9)

    sub = jax.lax.broadcasted_iota(jnp.int32, (_N, _N), 0)
    lan = jax.lax.broadcasted_iota(jnp.int32, (_N, _N), 1)
    lower = sub < lan
    Mf = jnp.where(lower & (iou > IOU_THRES), 1.0, 0.0)

    alive = jnp.where(scr > CONF_THRES, 1.0, 0.0)

    def cond(carry):
        _, changed = carry
        return changed

    def body(carry):
        S, _ = carry
        supp = jnp.dot(S, Mf, preferred_element_type=jnp.float32)
        S_new = alive * jnp.where(supp < 0.5, 1.0, 0.0)
        return S_new, jnp.any(S_new != S)

    keep, _ = jax.lax.while_loop(cond, body, (alive, jnp.bool_(True)))

    LTf = jnp.where(lower, 1.0, 0.0)
    rank = jnp.dot(keep, LTf, preferred_element_type=jnp.float32)

    rsub = jax.lax.broadcasted_iota(jnp.int32, (_R, _N), 0)
    PT = jnp.where((rank.astype(jnp.int32) == rsub) & (keep > 0.5), 1.0, 0.0)

    zc = jnp.zeros_like(xc)
    Dcol = jnp.concatenate(
        [xc - wc / 2.0, yc - hc / 2.0, xc + wc / 2.0, yc + hc / 2.0,
         dt[:, 4:5], clc, zc, zc], axis=1)

    out_ref[0] = jnp.dot(PT, Dcol, preferred_element_type=jnp.float32)


def _run_nms(d, dt):
    B = d.shape[0]
    return pl.pallas_call(
        _nms_body,
        grid=(B,),
        in_specs=[pl.BlockSpec((1, 8, _N), lambda b: (b, 0, 0)),
                  pl.BlockSpec((1, _N, 8), lambda b: (b, 0, 0))],
        out_specs=pl.BlockSpec((1, _R, 8), lambda b: (b, 0, 0)),
        out_shape=jax.ShapeDtypeStruct((B, _R, 8), jnp.float32),
    )(d, dt)


def kernel(x):
    pred = x[0]
    B, N, _ = pred.shape
    scores, cls_id = _compute_scores(pred)

    sc, idx = jax.lax.top_k(scores, MAX_NMS)
    xywh = jnp.take_along_axis(pred[..., :4], idx[..., None], axis=1)
    csel = jnp.take_along_axis(cls_id, idx, axis=1)

    pad = _N - MAX_NMS
    xywh = jnp.pad(xywh, ((0, 0), (0, pad), (0, 0)))
    sc = jnp.pad(sc, ((0, 0), (0, pad)), constant_values=-1.0)
    csel = jnp.pad(csel, ((0, 0), (0, pad)))

    dt = jnp.concatenate(
        [xywh, sc[..., None], csel[..., None],
         jnp.zeros((B, _N, 2), jnp.float32)], axis=2)
    d = jnp.transpose(dt, (0, 2, 1))

    out = _run_nms(d, dt)
    return out[:, :MAX_DET, :6]


# speedup vs baseline: 8.7779x; 1.0195x over previous
import jax
import jax.numpy as jnp
from jax.experimental import pallas as pl

CONF_THRES = 0.25
IOU_THRES = 0.45
MAX_NMS = 1000
MAX_DET = 300
MAX_WH = 4096.0

_N = 1024
_R = 512


def _scores_body(pred_ref, scores_ref, cls_ref):
    blk = pred_ref[0]
    rows = blk.shape[0]
    obj = blk[:, 4:5]
    prod = blk * obj
    lane = jax.lax.broadcasted_iota(jnp.int32, (rows, 85), 1)
    masked = jnp.where(lane >= 5, prod, -jnp.inf)
    conf = jnp.max(masked, axis=1, keepdims=True)
    cand = jnp.where(masked >= conf,
                     lane.astype(jnp.float32), 1e9)
    cls_id = jnp.min(cand, axis=1) - 5.0
    confv = conf[:, 0]
    scores_ref[0, 0, :] = jnp.where(confv > CONF_THRES, confv, -1.0)
    cls_ref[0, 0, :] = cls_id


def _compute_scores(pred):
    B, N, C = pred.shape
    rows = 4000
    nr = N // rows
    grid = (B, nr)
    scores, cls_id = pl.pallas_call(
        _scores_body,
        grid=grid,
        in_specs=[pl.BlockSpec((1, rows, C), lambda b, r: (b, r, 0))],
        out_specs=[pl.BlockSpec((1, 1, rows), lambda b, r: (b * nr + r, 0, 0)),
                   pl.BlockSpec((1, 1, rows), lambda b, r: (b * nr + r, 0, 0))],
        out_shape=[jax.ShapeDtypeStruct((B * nr, 1, rows), jnp.float32),
                   jax.ShapeDtypeStruct((B * nr, 1, rows), jnp.float32)],
    )(pred)
    return scores.reshape(B, N), cls_id.reshape(B, N)


def _nms_body(xywh_ref, sccls_ref, out_ref):
    dt = xywh_ref[0]
    sr = sccls_ref[0]

    sub = jax.lax.broadcasted_iota(jnp.int32, (_N, _N), 0)
    lan = jax.lax.broadcasted_iota(jnp.int32, (_N, _N), 1)
    eq = sub == lan

    def to_row(c):
        return jnp.sum(jnp.where(eq, c, 0.0), axis=0, keepdims=True)

    def to_col(r):
        return jnp.sum(jnp.where(eq, r, 0.0), axis=1, keepdims=True)

    xr = to_row(dt[:, 0:1])
    yr = to_row(dt[:, 1:2])
    wr = to_row(dt[:, 2:3])
    hr = to_row(dt[:, 3:4])
    scr, clr = sr[0:1, :], sr[1:2, :]
    offr = clr * MAX_WH
    rx1 = (xr - wr / 2.0) + offr
    ry1 = (yr - hr / 2.0) + offr
    rx2 = (xr + wr / 2.0) + offr
    ry2 = (yr + hr / 2.0) + offr
    area_r = (rx2 - rx1) * (ry2 - ry1)

    xc, yc, wc, hc = dt[:, 0:1], dt[:, 1:2], dt[:, 2:3], dt[:, 3:4]
    scc = to_col(scr)
    clc = to_col(clr)
    offc = clc * MAX_WH
    cx1 = (xc - wc / 2.0) + offc
    cy1 = (yc - hc / 2.0) + offc
    cx2 = (xc + wc / 2.0) + offc
    cy2 = (yc + hc / 2.0) + offc
    area_c = (cx2 - cx1) * (cy2 - cy1)

    iw = jnp.clip(jnp.minimum(cx2, rx2) - jnp.maximum(cx1, rx1), 0.0, None)
    ih = jnp.clip(jnp.minimum(cy2, ry2) - jnp.maximum(cy1, ry1), 0.0, None)
    inter = iw * ih
    iou = inter / (area_c + area_r - inter + 1e-9)

    lower = sub < lan
    Mf = jnp.where(lower & (iou > IOU_THRES), 1.0, 0.0)

    alive = jnp.where(scr > CONF_THRES, 1.0, 0.0)

    def cond(carry):
        _, changed = carry
        return changed

    def body(carry):
        S, _ = carry
        supp = jnp.dot(S, Mf, preferred_element_type=jnp.float32)
        S_new = alive * jnp.where(supp < 0.5, 1.0, 0.0)
        return S_new, jnp.any(S_new != S)

    keep, _ = jax.lax.while_loop(cond, body, (alive, jnp.bool_(True)))

    LTf = jnp.where(lower, 1.0, 0.0)
    rank = jnp.dot(keep, LTf, preferred_element_type=jnp.float32)

    rsub = jax.lax.broadcasted_iota(jnp.int32, (_R, _N), 0)
    PT = jnp.where((rank.astype(jnp.int32) == rsub) & (keep > 0.5), 1.0, 0.0)

    zc = jnp.zeros_like(xc)
    Dcol = jnp.concatenate(
        [xc - wc / 2.0, yc - hc / 2.0, xc + wc / 2.0, yc + hc / 2.0,
         scc, clc, zc, zc], axis=1)

    out_ref[0] = jnp.dot(PT, Dcol, preferred_element_type=jnp.float32)


def _run_nms(xywh, sccls):
    B = xywh.shape[0]
    return pl.pallas_call(
        _nms_body,
        grid=(B,),
        in_specs=[pl.BlockSpec((1, _N, 4), lambda b: (b, 0, 0)),
                  pl.BlockSpec((1, 2, _N), lambda b: (b, 0, 0))],
        out_specs=pl.BlockSpec((1, _R, 8), lambda b: (b, 0, 0)),
        out_shape=jax.ShapeDtypeStruct((B, _R, 8), jnp.float32),
    )(xywh, sccls)


def kernel(x):
    pred = x[0]
    B, N, _ = pred.shape
    scores, cls_id = _compute_scores(pred)

    sc, idx = jax.lax.top_k(scores, MAX_NMS)
    xywh = jnp.take_along_axis(pred[..., :4], idx[..., None], axis=1)
    csel = jnp.take_along_axis(cls_id, idx, axis=1)

    pad = _N - MAX_NMS
    xywh = jnp.pad(xywh, ((0, 0), (0, pad), (0, 0)))
    sc = jnp.pad(sc, ((0, 0), (0, pad)), constant_values=-1.0)
    csel = jnp.pad(csel, ((0, 0), (0, pad)))
    sccls = jnp.stack([sc, csel], axis=1)

    out = _run_nms(xywh, sccls)
    return out[:, :MAX_DET, :6]


# speedup vs baseline: 11.1282x; 1.2678x over previous
import jax
import jax.numpy as jnp
from jax.experimental import pallas as pl

CONF_THRES = 0.25
IOU_THRES = 0.45
MAX_NMS = 1000
MAX_DET = 300
MAX_WH = 4096.0

_N = 1024
_R = 512


def _scores_body(pred_ref, scores_ref, cls_ref):
    blk = pred_ref[0]
    rows = blk.shape[0]
    obj = blk[:, 4:5]
    prod = blk * obj
    lane = jax.lax.broadcasted_iota(jnp.int32, (rows, 85), 1)
    masked = jnp.where(lane >= 5, prod, -jnp.inf)
    conf = jnp.max(masked, axis=1, keepdims=True)
    cand = jnp.where(masked >= conf,
                     lane.astype(jnp.float32), 1e9)
    cls_id = jnp.min(cand, axis=1) - 5.0
    confv = conf[:, 0]
    scores_ref[0, 0, :] = jnp.where(confv > CONF_THRES, confv, -1.0)
    cls_ref[0, 0, :] = cls_id


def _compute_scores(pred):
    B, N, C = pred.shape
    rows = 4000
    nr = N // rows
    grid = (B, nr)
    scores, cls_id = pl.pallas_call(
        _scores_body,
        grid=grid,
        in_specs=[pl.BlockSpec((1, rows, C), lambda b, r: (b, r, 0))],
        out_specs=[pl.BlockSpec((1, 1, rows), lambda b, r: (b * nr + r, 0, 0)),
                   pl.BlockSpec((1, 1, rows), lambda b, r: (b * nr + r, 0, 0))],
        out_shape=[jax.ShapeDtypeStruct((B * nr, 1, rows), jnp.float32),
                   jax.ShapeDtypeStruct((B * nr, 1, rows), jnp.float32)],
    )(pred)
    return (scores.reshape(B, nr, rows), cls_id.reshape(B, nr, rows))


def _nms_body(xywh_ref, sccls_ref, out_ref):
    dt = xywh_ref[0]
    sr = sccls_ref[0]

    sub = jax.lax.broadcasted_iota(jnp.int32, (_N, _N), 0)
    lan = jax.lax.broadcasted_iota(jnp.int32, (_N, _N), 1)
    eq = sub == lan

    def to_row(c):
        return jnp.sum(jnp.where(eq, c, 0.0), axis=0, keepdims=True)

    def to_col(r):
        return jnp.sum(jnp.where(eq, r, 0.0), axis=1, keepdims=True)

    xr = to_row(dt[:, 0:1])
    yr = to_row(dt[:, 1:2])
    wr = to_row(dt[:, 2:3])
    hr = to_row(dt[:, 3:4])
    scr, clr = sr[0:1, :], sr[1:2, :]
    offr = clr * MAX_WH
    rx1 = (xr - wr / 2.0) + offr
    ry1 = (yr - hr / 2.0) + offr
    rx2 = (xr + wr / 2.0) + offr
    ry2 = (yr + hr / 2.0) + offr
    area_r = (rx2 - rx1) * (ry2 - ry1)

    xc, yc, wc, hc = dt[:, 0:1], dt[:, 1:2], dt[:, 2:3], dt[:, 3:4]
    scc = to_col(scr)
    clc = to_col(clr)
    offc = clc * MAX_WH
    cx1 = (xc - wc / 2.0) + offc
    cy1 = (yc - hc / 2.0) + offc
    cx2 = (xc + wc / 2.0) + offc
    cy2 = (yc + hc / 2.0) + offc
    area_c = (cx2 - cx1) * (cy2 - cy1)

    iw = jnp.clip(jnp.minimum(cx2, rx2) - jnp.maximum(cx1, rx1), 0.0, None)
    ih = jnp.clip(jnp.minimum(cy2, ry2) - jnp.maximum(cy1, ry1), 0.0, None)
    inter = iw * ih
    iou = inter / (area_c + area_r - inter + 1e-9)

    lower = sub < lan
    Mf = jnp.where(lower & (iou > IOU_THRES), 1.0, 0.0)

    alive = jnp.where(scr > CONF_THRES, 1.0, 0.0)

    def cond(carry):
        _, changed = carry
        return changed

    def body(carry):
        S, _ = carry
        supp = jnp.dot(S, Mf, preferred_element_type=jnp.float32)
        S_new = alive * jnp.where(supp < 0.5, 1.0, 0.0)
        return S_new, jnp.any(S_new != S)

    keep, _ = jax.lax.while_loop(cond, body, (alive, jnp.bool_(True)))

    LTf = jnp.where(lower, 1.0, 0.0)
    rank = jnp.dot(keep, LTf, preferred_element_type=jnp.float32)

    rsub = jax.lax.broadcasted_iota(jnp.int32, (_R, _N), 0)
    PT = jnp.where((rank.astype(jnp.int32) == rsub) & (keep > 0.5), 1.0, 0.0)

    zc = jnp.zeros_like(xc)
    Dcol = jnp.concatenate(
        [xc - wc / 2.0, yc - hc / 2.0, xc + wc / 2.0, yc + hc / 2.0,
         scc, clc, zc, zc], axis=1)

    out_ref[0] = jnp.dot(PT, Dcol, preferred_element_type=jnp.float32)


def _run_nms(xywh, sccls):
    B = xywh.shape[0]
    return pl.pallas_call(
        _nms_body,
        grid=(B,),
        in_specs=[pl.BlockSpec((1, _N, 4), lambda b: (b, 0, 0)),
                  pl.BlockSpec((1, 2, _N), lambda b: (b, 0, 0))],
        out_specs=pl.BlockSpec((1, _R, 8), lambda b: (b, 0, 0)),
        out_shape=jax.ShapeDtypeStruct((B, _R, 8), jnp.float32),
    )(xywh, sccls)


def kernel(x):
    pred = x[0]
    B, N, _ = pred.shape
    scores, cls_id = _compute_scores(pred)
    nr, rows = scores.shape[1], scores.shape[2]

    v1, li = jax.lax.top_k(scores, MAX_NMS)
    c1 = jnp.take_along_axis(cls_id, li, axis=2)
    g1 = li + (jnp.arange(nr, dtype=li.dtype) * rows)[None, :, None]
    vm = v1.reshape(B, nr * MAX_NMS)
    cm = c1.reshape(B, nr * MAX_NMS)
    gm = g1.reshape(B, nr * MAX_NMS)

    sc, i2 = jax.lax.top_k(vm, MAX_NMS)
    idx = jnp.take_along_axis(gm, i2, axis=1)
    csel = jnp.take_along_axis(cm, i2, axis=1)
    xywh = jnp.take_along_axis(pred[..., :4], idx[..., None], axis=1)

    pad = _N - MAX_NMS
    xywh = jnp.pad(xywh, ((0, 0), (0, pad), (0, 0)))
    sc = jnp.pad(sc, ((0, 0), (0, pad)), constant_values=-1.0)
    csel = jnp.pad(csel, ((0, 0), (0, pad)))
    sccls = jnp.stack([sc, csel], axis=1)

    out = _run_nms(xywh, sccls)
    return out[:, :MAX_DET, :6]


# speedup vs baseline: 11.5068x; 1.0340x over previous
import jax
import jax.numpy as jnp
from jax.experimental import pallas as pl

CONF_THRES = 0.25
IOU_THRES = 0.45
MAX_NMS = 1000
MAX_DET = 300
MAX_WH = 4096.0

_N = 1024
_R = 512


_ROWS = 4000
_NR = 5


def _scores_body(pred_ref, scores_ref, cls_ref):
    for r in range(_NR):
        blk = pred_ref[0, pl.ds(r * _ROWS, _ROWS), :]
        obj = blk[:, 4:5]
        prod = blk * obj
        lane = jax.lax.broadcasted_iota(jnp.int32, (_ROWS, 85), 1)
        masked = jnp.where(lane >= 5, prod, -jnp.inf)
        conf = jnp.max(masked, axis=1, keepdims=True)
        cand = jnp.where(masked >= conf,
                         lane.astype(jnp.float32), 1e9)
        cls_id = jnp.min(cand, axis=1) - 5.0
        confv = conf[:, 0]
        scores_ref[0, r, :] = jnp.where(confv > CONF_THRES, confv, -1.0)
        cls_ref[0, r, :] = cls_id


def _compute_scores(pred):
    B, N, C = pred.shape
    return pl.pallas_call(
        _scores_body,
        grid=(B,),
        in_specs=[pl.BlockSpec((1, N, C), lambda b: (b, 0, 0))],
        out_specs=[pl.BlockSpec((1, _NR, _ROWS), lambda b: (b, 0, 0)),
                   pl.BlockSpec((1, _NR, _ROWS), lambda b: (b, 0, 0))],
        out_shape=[jax.ShapeDtypeStruct((B, _NR, _ROWS), jnp.float32),
                   jax.ShapeDtypeStruct((B, _NR, _ROWS), jnp.float32)],
    )(pred)


def _nms_body(xywh_ref, sccls_ref, out_ref):
    dt = xywh_ref[0]
    sr = sccls_ref[0]

    sub = jax.lax.broadcasted_iota(jnp.int32, (_N, _N), 0)
    lan = jax.lax.broadcasted_iota(jnp.int32, (_N, _N), 1)
    eq = sub == lan

    def to_row(c):
        return jnp.sum(jnp.where(eq, c, 0.0), axis=0, keepdims=True)

    def to_col(r):
        return jnp.sum(jnp.where(eq, r, 0.0), axis=1, keepdims=True)

    xr = to_row(dt[:, 0:1])
    yr = to_row(dt[:, 1:2])
    wr = to_row(dt[:, 2:3])
    hr = to_row(dt[:, 3:4])
    scr, clr = sr[0:1, :], sr[1:2, :]
    offr = clr * MAX_WH
    rx1 = (xr - wr / 2.0) + offr
    ry1 = (yr - hr / 2.0) + offr
    rx2 = (xr + wr / 2.0) + offr
    ry2 = (yr + hr / 2.0) + offr
    area_r = (rx2 - rx1) * (ry2 - ry1)

    xc, yc, wc, hc = dt[:, 0:1], dt[:, 1:2], dt[:, 2:3], dt[:, 3:4]
    scc = to_col(scr)
    clc = to_col(clr)
    offc = clc * MAX_WH
    cx1 = (xc - wc / 2.0) + offc
    cy1 = (yc - hc / 2.0) + offc
    cx2 = (xc + wc / 2.0) + offc
    cy2 = (yc + hc / 2.0) + offc
    area_c = (cx2 - cx1) * (cy2 - cy1)

    iw = jnp.clip(jnp.minimum(cx2, rx2) - jnp.maximum(cx1, rx1), 0.0, None)
    ih = jnp.clip(jnp.minimum(cy2, ry2) - jnp.maximum(cy1, ry1), 0.0, None)
    inter = iw * ih
    iou = inter / (area_c + area_r - inter + 1e-9)

    lower = sub < lan
    Mf = jnp.where(lower & (iou > IOU_THRES), 1.0, 0.0)

    alive = jnp.where(scr > CONF_THRES, 1.0, 0.0)

    def cond(carry):
        _, changed = carry
        return changed

    def body(carry):
        S, _ = carry
        supp = jnp.dot(S, Mf, preferred_element_type=jnp.float32)
        S_new = alive * jnp.where(supp < 0.5, 1.0, 0.0)
        return S_new, jnp.any(S_new != S)

    keep, _ = jax.lax.while_loop(cond, body, (alive, jnp.bool_(True)))

    LTf = jnp.where(lower, 1.0, 0.0)
    rank = jnp.dot(keep, LTf, preferred_element_type=jnp.float32)

    rsub = jax.lax.broadcasted_iota(jnp.int32, (_R, _N), 0)
    PT = jnp.where((rank.astype(jnp.int32) == rsub) & (keep > 0.5), 1.0, 0.0)

    zc = jnp.zeros_like(xc)
    Dcol = jnp.concatenate(
        [xc - wc / 2.0, yc - hc / 2.0, xc + wc / 2.0, yc + hc / 2.0,
         scc, clc, zc, zc], axis=1)

    out_ref[0] = jnp.dot(PT, Dcol, preferred_element_type=jnp.float32)


def _run_nms(xywh, sccls):
    B = xywh.shape[0]
    return pl.pallas_call(
        _nms_body,
        grid=(B,),
        in_specs=[pl.BlockSpec((1, _N, 4), lambda b: (b, 0, 0)),
                  pl.BlockSpec((1, 2, _N), lambda b: (b, 0, 0))],
        out_specs=pl.BlockSpec((1, _R, 8), lambda b: (b, 0, 0)),
        out_shape=jax.ShapeDtypeStruct((B, _R, 8), jnp.float32),
    )(xywh, sccls)


def kernel(x):
    pred = x[0]
    B, N, _ = pred.shape
    scores, cls_id = _compute_scores(pred)
    nr, rows = scores.shape[1], scores.shape[2]

    v1, li = jax.lax.top_k(scores, MAX_NMS)
    c1 = jnp.take_along_axis(cls_id, li, axis=2)
    g1 = li + (jnp.arange(nr, dtype=li.dtype) * rows)[None, :, None]
    vm = v1.reshape(B, nr * MAX_NMS)
    cm = c1.reshape(B, nr * MAX_NMS)
    gm = g1.reshape(B, nr * MAX_NMS)

    sc, i2 = jax.lax.top_k(vm, MAX_NMS)
    idx = jnp.take_along_axis(gm, i2, axis=1)
    csel = jnp.take_along_axis(cm, i2, axis=1)
    xywh = jnp.take_along_axis(pred[..., :4], idx[..., None], axis=1)

    pad = _N - MAX_NMS
    xywh = jnp.pad(xywh, ((0, 0), (0, pad), (0, 0)))
    sc = jnp.pad(sc, ((0, 0), (0, pad)), constant_values=-1.0)
    csel = jnp.pad(csel, ((0, 0), (0, pad)))
    sccls = jnp.stack([sc, csel], axis=1)

    out = _run_nms(xywh, sccls)
    return out[:, :MAX_DET, :6]


# speedup vs baseline: 11.5714x; 1.0056x over previous
import jax
import jax.numpy as jnp
from jax.experimental import pallas as pl

CONF_THRES = 0.25
IOU_THRES = 0.45
MAX_NMS = 1000
MAX_DET = 300
MAX_WH = 4096.0

_N = 1024
_R = 512
_ROWS = 4000
_NR = 5


def _scores_body(pred_ref, scores_ref, cls_ref):
    for r in range(_NR):
        blk = pred_ref[0, pl.ds(r * _ROWS, _ROWS), :]
        obj = blk[:, 4:5]
        prod = blk * obj
        lane = jax.lax.broadcasted_iota(jnp.int32, (_ROWS, 85), 1)
        masked = jnp.where(lane >= 5, prod, -jnp.inf)
        conf = jnp.max(masked, axis=1, keepdims=True)
        cand = jnp.where(masked >= conf,
                         lane.astype(jnp.float32), 1e9)
        cls_id = jnp.min(cand, axis=1) - 5.0
        confv = conf[:, 0]
        scores_ref[0, r, :] = jnp.where(confv > CONF_THRES, confv, -1.0)
        cls_ref[0, r, :] = cls_id


def _compute_scores(pred):
    B, N, C = pred.shape
    spec = pl.BlockSpec((1, _NR, _ROWS), lambda b: (b, 0, 0))
    shp = jax.ShapeDtypeStruct((B, _NR, _ROWS), jnp.float32)
    return pl.pallas_call(
        _scores_body,
        grid=(B,),
        in_specs=[pl.BlockSpec((1, N, C), lambda b: (b, 0, 0))],
        out_specs=[spec] * 2,
        out_shape=[shp] * 2,
    )(pred)


def _nms_body(d_ref, out_ref):
    d = d_ref[0]

    sub = jax.lax.broadcasted_iota(jnp.int32, (_N, _N), 0)
    lan = jax.lax.broadcasted_iota(jnp.int32, (_N, _N), 1)
    eq = sub == lan

    def to_col(r):
        return jnp.sum(jnp.where(eq, r, 0.0), axis=1, keepdims=True)

    xr, yr, wr, hr = d[0:1, :], d[1:2, :], d[2:3, :], d[3:4, :]
    scr, clr = d[4:5, :], d[5:6, :]
    offr = clr * MAX_WH
    rx1 = (xr - wr / 2.0) + offr
    ry1 = (yr - hr / 2.0) + offr
    rx2 = (xr + wr / 2.0) + offr
    ry2 = (yr + hr / 2.0) + offr
    area_r = (rx2 - rx1) * (ry2 - ry1)

    xc, yc, wc, hc = to_col(xr), to_col(yr), to_col(wr), to_col(hr)
    scc, clc = to_col(scr), to_col(clr)
    offc = clc * MAX_WH
    cx1 = (xc - wc / 2.0) + offc
    cy1 = (yc - hc / 2.0) + offc
    cx2 = (xc + wc / 2.0) + offc
    cy2 = (yc + hc / 2.0) + offc
    area_c = (cx2 - cx1) * (cy2 - cy1)

    iw = jnp.clip(jnp.minimum(cx2, rx2) - jnp.maximum(cx1, rx1), 0.0, None)
    ih = jnp.clip(jnp.minimum(cy2, ry2) - jnp.maximum(cy1, ry1), 0.0, None)
    inter = iw * ih
    iou = inter / (area_c + area_r - inter + 1e-9)

    lower = sub < lan
    Mf = jnp.where(lower & (iou > IOU_THRES), 1.0, 0.0)

    alive = jnp.where(scr > CONF_THRES, 1.0, 0.0)

    def cond(carry):
        _, changed = carry
        return changed

    def body(carry):
        S, _ = carry
        supp = jnp.dot(S, Mf, preferred_element_type=jnp.float32)
        S_new = alive * jnp.where(supp < 0.5, 1.0, 0.0)
        return S_new, jnp.any(S_new != S)

    keep, _ = jax.lax.while_loop(cond, body, (alive, jnp.bool_(True)))

    LTf = jnp.where(lower, 1.0, 0.0)
    rank = jnp.dot(keep, LTf, preferred_element_type=jnp.float32)

    rsub = jax.lax.broadcasted_iota(jnp.int32, (_R, _N), 0)
    PT = jnp.where((rank.astype(jnp.int32) == rsub) & (keep > 0.5), 1.0, 0.0)

    zc = jnp.zeros_like(xc)
    Dcol = jnp.concatenate(
        [xc - wc / 2.0, yc - hc / 2.0, xc + wc / 2.0, yc + hc / 2.0,
         scc, clc, zc, zc], axis=1)

    out_ref[0] = jnp.dot(PT, Dcol, preferred_element_type=jnp.float32)


def _run_nms(d):
    B = d.shape[0]
    return pl.pallas_call(
        _nms_body,
        grid=(B,),
        in_specs=[pl.BlockSpec((1, 6, _N), lambda b: (b, 0, 0))],
        out_specs=pl.BlockSpec((1, _R, 8), lambda b: (b, 0, 0)),
        out_shape=jax.ShapeDtypeStruct((B, _R, 8), jnp.float32),
    )(d)


def kernel(x):
    pred = x[0]
    B, N, _ = pred.shape
    scores, cls_id = _compute_scores(pred)

    v1, li = jax.lax.top_k(scores, MAX_NMS)
    c1 = jnp.take_along_axis(cls_id, li, axis=2)
    g1 = li + (jnp.arange(_NR, dtype=li.dtype) * _ROWS)[None, :, None]
    vm = v1.reshape(B, _NR * MAX_NMS)
    cm = c1.reshape(B, _NR * MAX_NMS)
    gm = g1.reshape(B, _NR * MAX_NMS)

    sc, i2 = jax.lax.top_k(vm, MAX_NMS)
    idx = jnp.take_along_axis(gm, i2, axis=1)
    csel = jnp.take_along_axis(cm, i2, axis=1)
    rows = jnp.take_along_axis(pred, idx[..., None], axis=1)
    xs, ys, ws, hs = (rows[..., 0], rows[..., 1],
                      rows[..., 2], rows[..., 3])

    pad = _N - MAX_NMS
    sc = jnp.pad(sc, ((0, 0), (0, pad)), constant_values=-1.0)
    csel, xs, ys, ws, hs = [jnp.pad(a, ((0, 0), (0, pad)))
                            for a in (csel, xs, ys, ws, hs)]
    d = jnp.stack([xs, ys, ws, hs, sc, csel], axis=1)

    out = _run_nms(d)
    return out[:, :MAX_DET, :6]
